# baseline (speedup 1.0000x reference)
import functools

import jax
import jax.numpy as jnp
from jax import lax
from jax.experimental import pallas as pl
from jax.experimental.pallas import tpu as pltpu
from jax.experimental.pallas import tpu_sc as plsc

NUM_SEGMENTS = 1024
NG = 8
NCHUNK = 4
LANES = 16
CW = 32
BLK = 256
_SELU_ALPHA = 1.6732632423543772
_SELU_SCALE = 1.0507009873554805


def _selu(x):
    return _SELU_SCALE * jnp.where(x > 0, x, _SELU_ALPHA * (jnp.exp(x) - 1.0))


def _sc_seg_sum(x3, seg2, *, n):
    nblk = n // BLK
    tail = n - nblk * BLK
    nstage = (nblk + NG - 1) // NG
    mesh = plsc.VectorSubcoreMesh(core_axis_name="c", subcore_axis_name="s",
                                  num_cores=2, num_subcores=16)

    @functools.partial(
        pl.kernel,
        out_type=[
            jax.ShapeDtypeStruct((NG, NCHUNK, NUM_SEGMENTS * CW), jnp.float32),
            jax.ShapeDtypeStruct((NG, NUM_SEGMENTS), jnp.float32),
        ],
        mesh=mesh,
        compiler_params=pltpu.CompilerParams(needs_layout_passes=False),
        scratch_types=[
            pltpu.VMEM((BLK, CW), jnp.float32),
            pltpu.VMEM((BLK, CW), jnp.float32),
            pltpu.VMEM((BLK // LANES, LANES), jnp.int32),
            pltpu.VMEM((BLK // LANES, LANES), jnp.int32),
            pltpu.VMEM((NUM_SEGMENTS * CW,), jnp.float32),
            pltpu.VMEM((NUM_SEGMENTS,), jnp.float32),
            pltpu.SemaphoreType.DMA,
            pltpu.SemaphoreType.DMA,
        ],
    )
    def seg_sum(x_hbm, seg_hbm, sums_hbm, cnts_hbm,
                xb0, xb1, sb0, sb1, acc, cnt, sem0, sem1):
        wid = lax.axis_index("s") * 2 + lax.axis_index("c")
        g = wid // NCHUNK
        c = wid % NCHUNK

        iota = lax.iota(jnp.int32, LANES)
        zeros = jnp.zeros((LANES,), jnp.float32)
        ones = jnp.ones((LANES,), jnp.float32)
        cmask = lax.broadcast(c == 0, (LANES,))

        def zero_body(i, carry):
            acc[pl.ds(i * LANES, LANES)] = zeros
            return carry
        lax.fori_loop(0, NUM_SEGMENTS * CW // LANES, zero_body, 0)

        def zero_cnt(i, carry):
            cnt[pl.ds(i * LANES, LANES)] = zeros
            return carry
        lax.fori_loop(0, NUM_SEGMENTS // LANES, zero_cnt, 0)

        xbufs = (xb0, xb1)
        sbufs = (sb0, sb1)
        sems = (sem0, sem1)

        def mk(k, buf):
            b = g + NG * k
            cp1 = pltpu.make_async_copy(
                x_hbm.at[pl.ds(b * BLK, BLK), c], xbufs[buf], sems[buf])
            cp2 = pltpu.make_async_copy(
                seg_hbm.at[pl.ds(b * (BLK // LANES), BLK // LANES)],
                sbufs[buf], sems[buf])
            return cp1, cp2

        def active(k):
            return (g + NG * k) < nblk

        def process(buf, ngroups):
            xbuf = xbufs[buf]
            sbuf = sbufs[buf]

            def grp(gi, carry):
                seg16 = sbuf[gi, :]
                plsc.addupdate_scatter(cnt, [seg16], ones, mask=cmask)
                r0 = gi * LANES
                for j in range(LANES):
                    base = seg16[j] * CW
                    idx0 = iota + base
                    v0 = xbuf[r0 + j, pl.ds(0, LANES)]
                    plsc.addupdate_scatter(acc, [idx0], v0)
                    v1 = xbuf[r0 + j, pl.ds(LANES, LANES)]
                    plsc.addupdate_scatter(acc, [idx0 + LANES], v1)
                return carry
            lax.fori_loop(0, ngroups, grp, 0)

        def start_stage(k, buf):
            def go():
                cps = mk(k, buf)
                cps[0].start()
                cps[1].start()
            pl.when(active(k))(go)

        def finish_stage(k, buf):
            def go():
                cps = mk(k, buf)
                cps[0].wait()
                cps[1].wait()
                process(buf, BLK // LANES)
            pl.when(active(k))(go)

        npairs = (nstage + 1) // 2
        start_stage(0, 0)

        def pair(p, carry):
            k0 = 2 * p
            start_stage(k0 + 1, 1)
            finish_stage(k0, 0)
            start_stage(k0 + 2, 0)
            finish_stage(k0 + 1, 1)
            return carry
        lax.fori_loop(0, npairs, pair, 0)

        if tail:
            tgroups = tail // LANES
            tcp1 = pltpu.make_async_copy(
                x_hbm.at[pl.ds(nblk * BLK, tail), c],
                xb0.at[pl.ds(0, tail), :], sem0)
            tcp2 = pltpu.make_async_copy(
                seg_hbm.at[pl.ds(nblk * (BLK // LANES), tgroups)],
                sb0.at[pl.ds(0, tgroups), :], sem0)

            def tail_start():
                tcp1.start()
                tcp2.start()

            def tail_finish():
                tcp1.wait()
                tcp2.wait()
                process(0, tgroups)

            pl.when(g == 0)(tail_start)
            pl.when(g == 0)(tail_finish)

        pltpu.sync_copy(acc, sums_hbm.at[g, c])

        def flush_cnt():
            pltpu.sync_copy(cnt, cnts_hbm.at[g])
        pl.when(c == 0)(flush_cnt)

    return seg_sum(x3, seg2)


def _combine_mlp_kernel(sums_ref, cnts_ref, w1_ref, b1_ref, w2_ref, b2_ref,
                        out_ref):
    s = jnp.sum(sums_ref[...], axis=0)
    sums = jnp.concatenate([s[i] for i in range(NCHUNK)], axis=1)
    cnt = jnp.sum(cnts_ref[...], axis=0)[:, None]
    mean = sums / jnp.maximum(cnt, 1.0)
    h = _selu(jax.lax.dot_general(
        mean, w1_ref[...], (((1,), (0,)), ((), ())),
        preferred_element_type=jnp.float32,
        precision=jax.lax.Precision.HIGHEST) + b1_ref[...])
    out_ref[...] = jax.lax.dot_general(
        h, w2_ref[...], (((1,), (0,)), ((), ())),
        preferred_element_type=jnp.float32,
        precision=jax.lax.Precision.HIGHEST) + b2_ref[...]


def kernel(node_invariant_features, batch, W_pe, b_pe, W1, b1, W2, b2):
    x = node_invariant_features
    n, d = x.shape
    assert d == NCHUNK * CW and n % LANES == 0
    x3 = x.reshape(n, NCHUNK, CW)
    seg2 = batch.astype(jnp.int32).reshape(n // LANES, LANES)
    sums, cnts = _sc_seg_sum(x3, seg2, n=n)
    sums4 = sums.reshape(NG, NCHUNK, NUM_SEGMENTS, CW)

    b1r = b1.reshape(1, -1)
    b2r = b2.reshape(1, -1)
    out = pl.pallas_call(
        _combine_mlp_kernel,
        out_shape=jax.ShapeDtypeStruct((NUM_SEGMENTS, 1), jnp.float32),
    )(sums4, cnts, W1, b1r, W2, b2r)
    return out
